# baseline (speedup 1.0000x reference)
import functools

import jax
import jax.numpy as jnp
from jax import lax
from jax.experimental import pallas as pl
from jax.experimental.pallas import tpu as pltpu
from jax.experimental.pallas import tpu_sc as plsc


def _edge_pass(g, src, dst, inv, nstg):
    N, H = g.shape
    NWS, NCH2, CH = src.shape
    info = plsc.get_sparse_core_info()
    NC, NS, L = info.num_cores, info.num_subcores, info.num_lanes
    RV = H // L
    ZCH = 80
    NZCH = N // ZCH
    ZT = -(-NZCH // NS)
    NBUF = 4
    NHALF = nstg

    mesh = plsc.VectorSubcoreMesh(core_axis_name="c", subcore_axis_name="s")

    @functools.partial(
        pl.kernel,
        mesh=mesh,
        compiler_params=pltpu.CompilerParams(needs_layout_passes=False),
        out_type=jax.ShapeDtypeStruct((NC, N, H), jnp.float32),
        scratch_types=[
            pltpu.VMEM((NCH2, CH), jnp.int32),
            pltpu.VMEM((NCH2, CH), jnp.int32),
            pltpu.VMEM((NCH2, CH), jnp.float32),
            [pltpu.VMEM((CH, H), jnp.float32) for _ in range(NBUF)],
            pltpu.VMEM_SHARED((N, H), jnp.float32),
            [pltpu.SemaphoreType.DMA for _ in range(NBUF)],
            [pltpu.SemaphoreType.DMA for _ in range(NBUF)],
            pltpu.SemaphoreType.DMA,
        ],
    )
    def k(g_hbm, src_hbm, dst_hbm, inv_hbm, out_hbm,
          sidx, didx, invv, rows, acc, gsem, ssem, isem):
        cid = lax.axis_index("c")
        sid = lax.axis_index("s")
        wid = sid * NC + cid

        def zbody(i, _):
            rows[0][i // RV, pl.ds((i % RV) * L, L)] = (
                jnp.zeros((L,), jnp.float32))
            return 0
        lax.fori_loop(0, ZCH * RV, zbody, 0)
        for t in range(ZT):
            zc = sid + t * NS
            @pl.when(zc < NZCH)
            def _():
                pltpu.sync_copy(rows[0].at[pl.ds(0, ZCH)],
                                acc.at[pl.ds(zc * ZCH, ZCH)])
        plsc.subcore_barrier()

        def gather(cn, bi):
            return pltpu.async_copy(g_hbm.at[sidx.at[cn]], rows[bi], gsem[bi])

        GA = 2

        def process(cn, bi):
            ob = (bi + GA) % NBUF
            @pl.when(cn >= NBUF - GA)
            def _():
                pltpu.make_async_copy(rows[ob], acc.at[didx.at[cn]],
                                      ssem[ob]).wait()

            @pl.when(cn + GA < NCH2)
            def _():
                gather(cn + GA, ob)

            pltpu.make_async_copy(g_hbm.at[sidx.at[cn]], rows[bi],
                                  gsem[bi]).wait()

            cvec = jnp.zeros((L,), jnp.int32) + cn
            UN = 10

            def scale(q, _):
                for u in range(UN):
                    e = q * UN + u
                    s = plsc.load_gather(
                        invv, [cvec, jnp.zeros((L,), jnp.int32) + e])
                    for j in range(RV):
                        rows[bi][e, pl.ds(j * L, L)] = (
                            rows[bi][e, pl.ds(j * L, L)] * s)
                return 0
            lax.fori_loop(0, CH // UN, scale, 0)

            pltpu.async_copy(rows[bi], acc.at[didx.at[cn]], ssem[bi],
                             add=True)

        def stage(half, _):
            a = pltpu.async_copy(src_hbm.at[wid * NHALF + half], sidx, isem)
            b = pltpu.async_copy(dst_hbm.at[wid * NHALF + half], didx, isem)
            c = pltpu.async_copy(inv_hbm.at[wid * NHALF + half], invv, isem)
            a.wait()
            b.wait()
            c.wait()

            for p in range(GA):
                gather(p, p)

            def body(i, _):
                for bb in range(NBUF):
                    process(i * NBUF + bb, bb)
                return 0
            lax.fori_loop(0, NCH2 // NBUF, body, 0)
            for q in range(NBUF - GA):
                cnq = NCH2 - (NBUF - GA) + q
                pltpu.make_async_copy(rows[cnq % NBUF],
                                      acc.at[didx.at[cnq]],
                                      ssem[cnq % NBUF]).wait()
            return 0
        lax.fori_loop(0, NHALF, stage, 0)
        plsc.subcore_barrier()

        for t in range(ZT):
            zc = sid + t * NS
            @pl.when(zc < NZCH)
            def _():
                pltpu.sync_copy(acc.at[pl.ds(zc * ZCH, ZCH)],
                                out_hbm.at[cid, pl.ds(zc * ZCH, ZCH)])

    return k(g, src, dst, inv)


def _matmul_t(h, W):
    N, D = h.shape
    H = W.shape[0]
    BN = 1000

    def body(h_ref, w_ref, o_ref):
        o_ref[...] = lax.dot_general(
            h_ref[...], w_ref[...], (((1,), (1,)), ((), ())),
            preferred_element_type=jnp.float32)

    return pl.pallas_call(
        body,
        grid=(N // BN,),
        in_specs=[pl.BlockSpec((BN, D), lambda i: (i, 0)),
                  pl.BlockSpec((H, D), lambda i: (0, 0))],
        out_specs=pl.BlockSpec((BN, H), lambda i: (i, 0)),
        out_shape=jax.ShapeDtypeStruct((N, H), jnp.float32),
    )(h, W)


def _recip(ea):
    E = ea.shape[0]
    ea2 = ea.reshape(E // 128, 128)

    def body(a_ref, o_ref):
        o_ref[...] = 1.0 / a_ref[...]

    out = pl.pallas_call(
        body,
        out_shape=jax.ShapeDtypeStruct(ea2.shape, jnp.float32),
    )(ea2)
    return out.reshape(E)


def _combine_mm(acc, b, W):
    _, N, H = acc.shape
    BN = 1000

    def body(a_ref, b_ref, w_ref, o_ref):
        hfeat = jnp.maximum(a_ref[0] + a_ref[1] + b_ref[...], 0.0)
        o_ref[...] = lax.dot_general(
            hfeat, w_ref[...], (((1,), (1,)), ((), ())),
            preferred_element_type=jnp.float32)

    return pl.pallas_call(
        body,
        grid=(N // BN,),
        in_specs=[pl.BlockSpec((2, BN, H), lambda i: (0, i, 0)),
                  pl.BlockSpec((1, H), lambda i: (0, 0)),
                  pl.BlockSpec((H, H), lambda i: (0, 0))],
        out_specs=pl.BlockSpec((BN, H), lambda i: (i, 0)),
        out_shape=jax.ShapeDtypeStruct((N, H), jnp.float32),
    )(acc, b.reshape(1, H), W)


def _combine_act(acc, b):
    _, N, H = acc.shape
    BN = 1000

    def body(a_ref, b_ref, o_ref):
        o_ref[...] = jnp.maximum(a_ref[0] + a_ref[1] + b_ref[...], 0.0)

    return pl.pallas_call(
        body,
        grid=(N // BN,),
        in_specs=[pl.BlockSpec((2, BN, H), lambda i: (0, i, 0)),
                  pl.BlockSpec((1, H), lambda i: (0, 0))],
        out_specs=pl.BlockSpec((BN, H), lambda i: (i, 0)),
        out_shape=jax.ShapeDtypeStruct((N, H), jnp.float32),
    )(acc, b.reshape(1, H))


def kernel(x, edge_index, edge_attr, W1, b1, W2, b2):
    E = edge_attr.shape[0]
    info = plsc.get_sparse_core_info()
    NW = info.num_cores * info.num_subcores
    EPT = E // NW
    CH = 50
    NCH = EPT // CH
    NSTG = 5
    NCHS = NCH // NSTG

    src = edge_index[0].reshape(NW * NSTG, NCHS, CH)
    dst = edge_index[1].reshape(NW * NSTG, NCHS, CH)
    inv = _recip(edge_attr).reshape(NW * NSTG, NCHS, CH)

    g1 = _matmul_t(x, W1)
    acc1 = _edge_pass(g1, src, dst, inv, NSTG)
    g2 = _combine_mm(acc1, b1, W2)
    acc2 = _edge_pass(g2, src, dst, inv, NSTG)
    return _combine_act(acc2, b2)


# speedup vs baseline: 2.3851x; 2.3851x over previous
import functools

import jax
import jax.numpy as jnp
from jax import lax
from jax.experimental import pallas as pl
from jax.experimental.pallas import tpu as pltpu
from jax.experimental.pallas import tpu_sc as plsc


def _edge_pass(g, src, dst, inv, nstg):
    N, H = g.shape
    NWS, NCH2, CH = src.shape
    info = plsc.get_sparse_core_info()
    NC, NS, L = info.num_cores, info.num_subcores, info.num_lanes
    RV = H // L
    ZCH = 80
    NZCH = N // ZCH
    ZT = -(-NZCH // NS)
    NBUF = 4
    NHALF = nstg

    mesh = plsc.VectorSubcoreMesh(core_axis_name="c", subcore_axis_name="s")

    @functools.partial(
        pl.kernel,
        mesh=mesh,
        compiler_params=pltpu.CompilerParams(needs_layout_passes=False),
        out_type=jax.ShapeDtypeStruct((NC, N, H), jnp.float32),
        scratch_types=[
            pltpu.VMEM((NCH2, CH), jnp.int32),
            pltpu.VMEM((NCH2, CH), jnp.int32),
            pltpu.VMEM((NCH2, CH), jnp.float32),
            [pltpu.VMEM((CH, H), jnp.float32) for _ in range(NBUF)],
            pltpu.VMEM_SHARED((N, H), jnp.float32),
            [pltpu.SemaphoreType.DMA for _ in range(NBUF)],
            [pltpu.SemaphoreType.DMA for _ in range(NBUF)],
            pltpu.SemaphoreType.DMA,
        ],
    )
    def k(g_hbm, src_hbm, dst_hbm, inv_hbm, out_hbm,
          sidx, didx, invv, rows, acc, gsem, ssem, isem):
        cid = lax.axis_index("c")
        sid = lax.axis_index("s")
        wid = sid * NC + cid

        def zbody(i, _):
            rows[0][i // RV, pl.ds((i % RV) * L, L)] = (
                jnp.zeros((L,), jnp.float32))
            return 0
        lax.fori_loop(0, ZCH * RV, zbody, 0)
        for t in range(ZT):
            zc = sid + t * NS
            @pl.when(zc < NZCH)
            def _():
                pltpu.sync_copy(rows[0].at[pl.ds(0, ZCH)],
                                acc.at[pl.ds(zc * ZCH, ZCH)])
        plsc.subcore_barrier()

        def gather(cn, bi):
            return pltpu.async_copy(g_hbm.at[sidx.at[cn]], rows[bi], gsem[bi])

        GA = 2

        def process(cn, bi):
            ob = (bi + GA) % NBUF
            @pl.when(cn >= NBUF - GA)
            def _():
                pltpu.make_async_copy(rows[ob], acc.at[didx.at[cn]],
                                      ssem[ob]).wait()

            @pl.when(cn + GA < NCH2)
            def _():
                gather(cn + GA, ob)

            pltpu.make_async_copy(g_hbm.at[sidx.at[cn]], rows[bi],
                                  gsem[bi]).wait()

            cvec = jnp.zeros((L,), jnp.int32) + cn
            UN = 5

            def scale(q, _):
                for u in range(UN):
                    e = q * UN + u
                    s = plsc.load_gather(
                        invv, [cvec, jnp.zeros((L,), jnp.int32) + e])
                    for j in range(RV):
                        rows[bi][e, pl.ds(j * L, L)] = (
                            rows[bi][e, pl.ds(j * L, L)] * s)
                return 0
            lax.fori_loop(0, CH // UN, scale, 0)

            pltpu.async_copy(rows[bi], acc.at[didx.at[cn]], ssem[bi],
                             add=True)

        def stage(half, _):
            a = pltpu.async_copy(src_hbm.at[wid * NHALF + half], sidx, isem)
            b = pltpu.async_copy(dst_hbm.at[wid * NHALF + half], didx, isem)
            c = pltpu.async_copy(inv_hbm.at[wid * NHALF + half], invv, isem)
            a.wait()
            b.wait()
            c.wait()

            for p in range(GA):
                gather(p, p)

            def body(i, _):
                for bb in range(NBUF):
                    process(i * NBUF + bb, bb)
                return 0
            lax.fori_loop(0, NCH2 // NBUF, body, 0)
            for q in range(NBUF - GA):
                cnq = NCH2 - (NBUF - GA) + q
                pltpu.make_async_copy(rows[cnq % NBUF],
                                      acc.at[didx.at[cnq]],
                                      ssem[cnq % NBUF]).wait()
            return 0
        lax.fori_loop(0, NHALF, stage, 0)
        plsc.subcore_barrier()

        for t in range(ZT):
            zc = sid + t * NS
            @pl.when(zc < NZCH)
            def _():
                pltpu.sync_copy(acc.at[pl.ds(zc * ZCH, ZCH)],
                                out_hbm.at[cid, pl.ds(zc * ZCH, ZCH)])

    return k(g, src, dst, inv)


def _matmul_t(h, W):
    N, D = h.shape
    H = W.shape[0]
    BN = 1000

    def body(h_ref, w_ref, o_ref):
        o_ref[...] = lax.dot_general(
            h_ref[...], w_ref[...], (((1,), (1,)), ((), ())),
            preferred_element_type=jnp.float32)

    return pl.pallas_call(
        body,
        grid=(N // BN,),
        in_specs=[pl.BlockSpec((BN, D), lambda i: (i, 0)),
                  pl.BlockSpec((H, D), lambda i: (0, 0))],
        out_specs=pl.BlockSpec((BN, H), lambda i: (i, 0)),
        out_shape=jax.ShapeDtypeStruct((N, H), jnp.float32),
    )(h, W)


def _recip(ea):
    E = ea.shape[0]
    ea2 = ea.reshape(E // 128, 128)

    def body(a_ref, o_ref):
        o_ref[...] = 1.0 / a_ref[...]

    out = pl.pallas_call(
        body,
        out_shape=jax.ShapeDtypeStruct(ea2.shape, jnp.float32),
    )(ea2)
    return out.reshape(E)


def _combine_mm(acc, b, W):
    _, N, H = acc.shape
    BN = 1000

    def body(a_ref, b_ref, w_ref, o_ref):
        hfeat = jnp.maximum(a_ref[0] + a_ref[1] + b_ref[...], 0.0)
        o_ref[...] = lax.dot_general(
            hfeat, w_ref[...], (((1,), (1,)), ((), ())),
            preferred_element_type=jnp.float32)

    return pl.pallas_call(
        body,
        grid=(N // BN,),
        in_specs=[pl.BlockSpec((2, BN, H), lambda i: (0, i, 0)),
                  pl.BlockSpec((1, H), lambda i: (0, 0)),
                  pl.BlockSpec((H, H), lambda i: (0, 0))],
        out_specs=pl.BlockSpec((BN, H), lambda i: (i, 0)),
        out_shape=jax.ShapeDtypeStruct((N, H), jnp.float32),
    )(acc, b.reshape(1, H), W)


def _combine_act(acc, b):
    _, N, H = acc.shape
    BN = 1000

    def body(a_ref, b_ref, o_ref):
        o_ref[...] = jnp.maximum(a_ref[0] + a_ref[1] + b_ref[...], 0.0)

    return pl.pallas_call(
        body,
        grid=(N // BN,),
        in_specs=[pl.BlockSpec((2, BN, H), lambda i: (0, i, 0)),
                  pl.BlockSpec((1, H), lambda i: (0, 0))],
        out_specs=pl.BlockSpec((BN, H), lambda i: (i, 0)),
        out_shape=jax.ShapeDtypeStruct((N, H), jnp.float32),
    )(acc, b.reshape(1, H))


def kernel(x, edge_index, edge_attr, W1, b1, W2, b2):
    E = edge_attr.shape[0]
    info = plsc.get_sparse_core_info()
    NW = info.num_cores * info.num_subcores
    EPT = E // NW
    CH = 50
    NCH = EPT // CH
    NSTG = 5
    NCHS = NCH // NSTG

    src = edge_index[0].reshape(NW * NSTG, NCHS, CH)
    dst = edge_index[1].reshape(NW * NSTG, NCHS, CH)
    inv = _recip(edge_attr).reshape(NW * NSTG, NCHS, CH)

    g1 = _matmul_t(x, W1)
    acc1 = _edge_pass(g1, src, dst, inv, NSTG)
    g2 = _combine_mm(acc1, b1, W2)
    acc2 = _edge_pass(g2, src, dst, inv, NSTG)
    return _combine_act(acc2, b2)
